# baseline (speedup 1.0000x reference)
import functools

import jax
import jax.numpy as jnp
from jax import lax
from jax.experimental import pallas as pl
from jax.experimental.pallas import tpu as pltpu
from jax.experimental.pallas import tpu_sc as plsc

D = 32
B = 16384
H = 50
N = B * H
NC = 2
NS = 16
NW = NC * NS
TBW = B // (128 * NW)
PANELS = H * TBW
BANK = 8
GROUPS = PANELS // BANK


def _transpose_panel(rows, out_v, i0, i1):
    def cbody(c, carry):
        v0 = rows[c, pl.ds(0, 16)]
        v1 = rows[c, pl.ds(16, 16)]
        plsc.store_scatter(out_v, [i0 + c], v0)
        plsc.store_scatter(out_v, [i1 + c], v1)
        return carry

    lax.fori_loop(0, 128, cbody, 0)


V = 1_000_000
FULL_TC = 7808
KPW = FULL_TC // NW
TAIL0 = FULL_TC * 128
TAILN = (V - TAIL0) * D


def _build_relayout():
    mesh = plsc.VectorSubcoreMesh(core_axis_name="c", subcore_axis_name="s")

    @functools.partial(
        pl.kernel,
        mesh=mesh,
        compiler_params=pltpu.CompilerParams(use_tc_tiling_on_sc=True,
                                             needs_layout_passes=False),
        out_type=jax.ShapeDtypeStruct((V * D,), jnp.float32),
        scratch_types=[
            pltpu.VMEM((D, 128), jnp.float32),
            pltpu.VMEM((D, 128), jnp.float32),
            pltpu.VMEM((128 * D,), jnp.float32),
            pltpu.VMEM((128 * D,), jnp.float32),
            pltpu.SemaphoreType.DMA,
            pltpu.SemaphoreType.DMA,
            pltpu.SemaphoreType.DMA,
            pltpu.SemaphoreType.DMA,
        ],
    )
    def relayout_kernel(table_t, tail, scratch, pan0, pan1, row0, row1,
                        si0, si1, so0, so1):
        wid = lax.axis_index("s") * NC + lax.axis_index("c")
        base = KPW * wid
        pans = (pan0, pan1)
        rows = (row0, row1)
        sins = (si0, si1)
        souts = (so0, so1)

        @pl.when(wid == 0)
        def _():
            def tbody(i, carry):
                pltpu.sync_copy(tail.at[pl.ds(i * 2048, 2048)],
                                row0.at[pl.ds(0, 2048)])
                pltpu.sync_copy(row0.at[pl.ds(0, 2048)],
                                scratch.at[pl.ds(TAIL0 * D + i * 2048,
                                                 2048)])
                return carry
            lax.fori_loop(0, TAILN // 2048, tbody, 0)

        ib = lax.iota(jnp.int32, 16)
        ibs = [(ib + 16 * jj) * D for jj in range(8)]

        def fire_in(k, b):
            pltpu.async_copy(
                table_t.at[:, pl.ds((base + k) * 128, 128)], pans[b],
                sins[b])

        def wait_in(k, b):
            pltpu.make_async_copy(
                table_t.at[:, pl.ds((base + k) * 128, 128)], pans[b],
                sins[b]).wait()

        def fire_out(k, b):
            pltpu.async_copy(rows[b],
                             scratch.at[pl.ds((base + k) * 128 * D,
                                              128 * D)], souts[b])

        def wait_out(k, b):
            pltpu.make_async_copy(
                rows[b],
                scratch.at[pl.ds((base + k) * 128 * D, 128 * D)],
                souts[b]).wait()

        def transpose(pan, row):
            def dbody(d, carry):
                for jj in range(8):
                    v = pan[d, pl.ds(16 * jj, 16)]
                    plsc.store_scatter(row, [ibs[jj] + d], v)
                return carry
            lax.fori_loop(0, D, dbody, 0)

        fire_in(0, 0)
        fire_in(1, 1)

        def body(kk, carry):
            for par in range(2):
                k = 2 * kk + par
                wait_in(k, par)

                @pl.when(kk >= 1)
                def _():
                    wait_out(k - 2, par)

                transpose(pans[par], rows[par])
                fire_out(k, par)

                @pl.when(k + 2 <= KPW - 1)
                def _():
                    fire_in(k + 2, par)

            return carry

        lax.fori_loop(0, KPW // 2, body, 0)
        wait_out(KPW - 2, 0)
        wait_out(KPW - 1, 1)

    return relayout_kernel


def _build():
    mesh = plsc.VectorSubcoreMesh(core_axis_name="c", subcore_axis_name="s")

    @functools.partial(
        pl.kernel,
        mesh=mesh,
        compiler_params=pltpu.CompilerParams(use_tc_tiling_on_sc=False,
                                             needs_layout_passes=False),
        out_type=jax.ShapeDtypeStruct((N * D,), jnp.float32),
        scratch_types=[
            pltpu.VMEM((H, 128 * TBW), jnp.int32),
            pltpu.VMEM((2 * BANK, 128, D), jnp.float32),
            pltpu.VMEM((128 * D,), jnp.float32),
            pltpu.VMEM((128 * D,), jnp.float32),
            pltpu.SemaphoreType.DMA,
            pltpu.SemaphoreType.DMA,
            pltpu.SemaphoreType.DMA,
            pltpu.SemaphoreType.DMA,
            pltpu.SemaphoreType.DMA,
        ],
    )
    def gather_kernel(idx_hbm, table_hbm, out_hbm, idx_v, rows_v, out_va,
                      out_vb, sga, sgb, sst0, sst1, sidx):
        wid = lax.axis_index("s") * NC + lax.axis_index("c")
        nbase = 128 * TBW * wid

        for h in range(H):
            pltpu.async_copy(
                idx_hbm.at[pl.ds(h * B + nbase, 128 * TBW)], idx_v.at[h],
                sidx)
        for h in range(H):
            pltpu.make_async_copy(
                idx_hbm.at[pl.ds(h * B + nbase, 128 * TBW)], idx_v.at[h],
                sidx).wait()

        ib = lax.iota(jnp.int32, 16)
        i0 = ib * 128
        i1 = i0 + (16 * 128)
        ssts = (sst0, sst1)
        obufs = (out_va, out_vb)

        def fire(p, buf, sem):
            h = p // TBW
            t = p % TBW
            pltpu.async_copy(
                table_hbm.at[idx_v.at[h, pl.ds(t * 128, 128)]],
                rows_v.at[buf], sem)

        def wait_gather(p, buf, sem):
            h = p // TBW
            t = p % TBW
            pltpu.make_async_copy(
                table_hbm.at[idx_v.at[h, pl.ds(t * 128, 128)]],
                rows_v.at[buf], sem).wait()

        def store(p, ob):
            h = p // TBW
            tb = TBW * wid + (p % TBW)
            for ta in range(4):
                ofs = ((h * 4 + ta) * (B // 128) + tb) * 1024
                pltpu.async_copy(obufs[ob].at[pl.ds(ta * 1024, 1024)],
                                 out_hbm.at[pl.ds(ofs, 1024)], ssts[ob])

        def wait_store(p, ob):
            h = p // TBW
            tb = TBW * wid + (p % TBW)
            for ta in range(4):
                ofs = ((h * 4 + ta) * (B // 128) + tb) * 1024
                pltpu.make_async_copy(
                    obufs[ob].at[pl.ds(ta * 1024, 1024)],
                    out_hbm.at[pl.ds(ofs, 1024)], ssts[ob]).wait()

        def process(p, buf, pipe):
            ob = buf % 2
            wait_gather(p, buf, sga if buf < BANK else sgb)

            if isinstance(pipe, int):
                if pipe >= 2:
                    wait_store(p - 2, ob)
            else:
                @pl.when(pipe >= 2)
                def _():
                    wait_store(p - 2, ob)

            _transpose_panel(rows_v.at[buf], obufs[ob], i0, i1)
            store(p, ob)

        for b in range(BANK):
            fire(b, b, sga)
        for b in range(BANK):
            fire(BANK + b, BANK + b, sgb)

        def body(gg, carry):
            ga = 2 * gg
            for b in range(BANK):
                process(ga * BANK + b, b, ga * BANK + b)

            @pl.when(ga + 2 <= GROUPS - 1)
            def _():
                for b in range(BANK):
                    fire((ga + 2) * BANK + b, b, sga)

            for b in range(BANK):
                process((ga + 1) * BANK + b, BANK + b,
                        (ga + 1) * BANK + b)

            @pl.when(ga + 3 <= GROUPS - 1)
            def _():
                for b in range(BANK):
                    fire((ga + 3) * BANK + b, BANK + b, sgb)

            return carry

        lax.fori_loop(0, (GROUPS - 1) // 2, body, 0)
        for b in range(BANK):
            process((GROUPS - 1) * BANK + b, b, (GROUPS - 1) * BANK + b)
        wait_store(PANELS - 2, 0)
        wait_store(PANELS - 1, 1)

    return gather_kernel


_GATHER = _build()
_RELAYOUT = _build_relayout()


def kernel(item_id, table):
    idx_t = jnp.transpose(item_id).reshape(-1).astype(jnp.int32)
    table_t = jnp.transpose(table)
    tail = lax.slice(table, (TAIL0, 0), (V, D)).reshape(-1)
    table_lin = _RELAYOUT(table_t, tail).reshape(V, D)
    out1d = _GATHER(idx_t, table_lin)
    out5 = out1d.reshape(H, 4, B // 128, 8, 128)
    return out5.transpose(2, 4, 0, 1, 3).reshape(B, H, D)


# speedup vs baseline: 1.1635x; 1.1635x over previous
import functools

import jax
import jax.numpy as jnp
from jax import lax
from jax.experimental import pallas as pl
from jax.experimental.pallas import tpu as pltpu
from jax.experimental.pallas import tpu_sc as plsc

D = 32
B = 16384
H = 50
N = B * H
NC = 2
NS = 16
NW = NC * NS
TBW = B // (128 * NW)
PANELS = H * TBW
BANK = 8
GROUPS = PANELS // BANK

V = 1_000_000
FULL_TC = 7808
KPW = FULL_TC // NW
TAIL0 = FULL_TC * 128
TAILN = (V - TAIL0) * D

RP = D + 1
CP = 129


def _build_relayout():
    mesh = plsc.VectorSubcoreMesh(core_axis_name="c", subcore_axis_name="s")

    @functools.partial(
        pl.kernel,
        mesh=mesh,
        compiler_params=pltpu.CompilerParams(use_tc_tiling_on_sc=True,
                                             needs_layout_passes=False),
        out_type=jax.ShapeDtypeStruct((V * D,), jnp.float32),
        scratch_types=[
            pltpu.VMEM((D, 128), jnp.float32),
            pltpu.VMEM((D, 128), jnp.float32),
            pltpu.VMEM((128 * RP,), jnp.float32),
            pltpu.VMEM((128 * D,), jnp.float32),
            pltpu.VMEM((128 * D,), jnp.float32),
            pltpu.SemaphoreType.DMA,
            pltpu.SemaphoreType.DMA,
            pltpu.SemaphoreType.DMA,
            pltpu.SemaphoreType.DMA,
        ],
    )
    def relayout_kernel(table_t, tail, scratch, pan0, pan1, row_p, row0,
                        row1, si0, si1, so0, so1):
        wid = lax.axis_index("s") * NC + lax.axis_index("c")
        base = KPW * wid
        pans = (pan0, pan1)
        rows = (row0, row1)
        sins = (si0, si1)
        souts = (so0, so1)

        @pl.when(wid == 0)
        def _():
            def tbody(i, carry):
                pltpu.sync_copy(tail.at[pl.ds(i * 2048, 2048)],
                                row0.at[pl.ds(0, 2048)])
                pltpu.sync_copy(row0.at[pl.ds(0, 2048)],
                                scratch.at[pl.ds(TAIL0 * D + i * 2048,
                                                 2048)])
                return carry
            lax.fori_loop(0, TAILN // 2048, tbody, 0)

        ib = lax.iota(jnp.int32, 16)
        ibs = [(ib + 16 * jj) * RP for jj in range(8)]

        def fire_in(k, b):
            pltpu.async_copy(
                table_t.at[:, pl.ds((base + k) * 128, 128)], pans[b],
                sins[b])

        def wait_in(k, b):
            pltpu.make_async_copy(
                table_t.at[:, pl.ds((base + k) * 128, 128)], pans[b],
                sins[b]).wait()

        def fire_out(k, b):
            pltpu.async_copy(rows[b],
                             scratch.at[pl.ds((base + k) * 128 * D,
                                              128 * D)], souts[b])

        def wait_out(k, b):
            pltpu.make_async_copy(
                rows[b],
                scratch.at[pl.ds((base + k) * 128 * D, 128 * D)],
                souts[b]).wait()

        def transpose(pan, row):
            def dbody(dd, carry):
                for du in range(4):
                    d = dd * 4 + du
                    dvec = jnp.full((16,), 0, jnp.int32) + d
                    for jj in range(8):
                        v = pan[d, pl.ds(16 * jj, 16)]
                        plsc.store_scatter(row_p, [ibs[jj] + dvec], v)
                return carry
            lax.fori_loop(0, 8, dbody, 0)

            def cbody(cc, carry):
                for cl in range(8):
                    c = cc * 8 + cl
                    row[pl.ds(c * D, 16)] = row_p[pl.ds(c * RP, 16)]
                    row[pl.ds(c * D + 16, 16)] = (
                        row_p[pl.ds(c * RP + 16, 16)])
                return carry
            lax.fori_loop(0, 16, cbody, 0)

        fire_in(0, 0)
        fire_in(1, 1)

        def body(kk, carry):
            for par in range(2):
                k = 2 * kk + par
                wait_in(k, par)

                @pl.when(kk >= 1)
                def _():
                    wait_out(k - 2, par)

                transpose(pans[par], rows[par])
                fire_out(k, par)

                @pl.when(k + 2 <= KPW - 1)
                def _():
                    fire_in(k + 2, par)

            return carry

        lax.fori_loop(0, KPW // 2, body, 0)
        wait_out(KPW - 2, 0)
        wait_out(KPW - 1, 1)

    return relayout_kernel


def _build_gather():
    mesh = plsc.VectorSubcoreMesh(core_axis_name="c", subcore_axis_name="s")

    @functools.partial(
        pl.kernel,
        mesh=mesh,
        compiler_params=pltpu.CompilerParams(use_tc_tiling_on_sc=False,
                                             needs_layout_passes=False),
        out_type=jax.ShapeDtypeStruct((N * D,), jnp.float32),
        scratch_types=[
            pltpu.VMEM((H, 128 * TBW), jnp.int32),
            pltpu.VMEM((2 * BANK, 128, D), jnp.float32),
            pltpu.VMEM((128 * D,), jnp.float32),
            pltpu.VMEM((128 * D,), jnp.float32),
            pltpu.SemaphoreType.DMA,
            pltpu.SemaphoreType.DMA,
            pltpu.SemaphoreType.DMA,
            pltpu.SemaphoreType.DMA,
            pltpu.SemaphoreType.DMA,
        ],
    )
    def gather_kernel(idx_hbm, table_hbm, out_hbm, idx_v, rows_v,
                      out_va, out_vb, sga, sgb, sst0, sst1, sidx):
        wid = lax.axis_index("s") * NC + lax.axis_index("c")
        nbase = 128 * TBW * wid

        for h in range(H):
            pltpu.async_copy(
                idx_hbm.at[pl.ds(h * B + nbase, 128 * TBW)], idx_v.at[h],
                sidx)
        for h in range(H):
            pltpu.make_async_copy(
                idx_hbm.at[pl.ds(h * B + nbase, 128 * TBW)], idx_v.at[h],
                sidx).wait()

        ib = lax.iota(jnp.int32, 16)
        i0p = ib * 128
        i1p = i0p + 16 * 128
        ssts = (sst0, sst1)
        obufs = (out_va, out_vb)

        def fire(p, buf, sem):
            h = p // TBW
            t = p % TBW
            pltpu.async_copy(
                table_hbm.at[idx_v.at[h, pl.ds(t * 128, 128)]],
                rows_v.at[buf], sem)

        def wait_gather(p, buf, sem):
            h = p // TBW
            t = p % TBW
            pltpu.make_async_copy(
                table_hbm.at[idx_v.at[h, pl.ds(t * 128, 128)]],
                rows_v.at[buf], sem).wait()

        def store(p, ob):
            h = p // TBW
            tb = TBW * wid + (p % TBW)
            for ta in range(4):
                ofs = ((h * 4 + ta) * (B // 128) + tb) * 1024
                pltpu.async_copy(obufs[ob].at[pl.ds(ta * 1024, 1024)],
                                 out_hbm.at[pl.ds(ofs, 1024)], ssts[ob])

        def wait_store(p, ob):
            h = p // TBW
            tb = TBW * wid + (p % TBW)
            for ta in range(4):
                ofs = ((h * 4 + ta) * (B // 128) + tb) * 1024
                pltpu.make_async_copy(
                    obufs[ob].at[pl.ds(ta * 1024, 1024)],
                    out_hbm.at[pl.ds(ofs, 1024)], ssts[ob]).wait()

        def transpose_panel(rows, out_c):
            def cbody(cc, carry):
                for cl16 in range(16):
                    c = cc * 16 + cl16
                    cvec = jnp.full((16,), 0, jnp.int32) + c
                    v0 = rows[c, pl.ds(0, 16)]
                    v1 = rows[c, pl.ds(16, 16)]
                    plsc.store_scatter(out_c, [i0p + cvec], v0)
                    plsc.store_scatter(out_c, [i1p + cvec], v1)
                return carry
            lax.fori_loop(0, 8, cbody, 0)

        def process(p, buf, pipe):
            ob = buf % 2
            wait_gather(p, buf, sga if buf < BANK else sgb)

            if isinstance(pipe, int):
                if pipe >= 2:
                    wait_store(p - 2, ob)
            else:
                @pl.when(pipe >= 2)
                def _():
                    wait_store(p - 2, ob)

            transpose_panel(rows_v.at[buf], obufs[ob])
            store(p, ob)

        for b in range(BANK):
            fire(b, b, sga)
        for b in range(BANK):
            fire(BANK + b, BANK + b, sgb)

        def body(gg, carry):
            ga = 2 * gg
            for b in range(BANK):
                process(ga * BANK + b, b, ga * BANK + b)

            @pl.when(ga + 2 <= GROUPS - 1)
            def _():
                for b in range(BANK):
                    fire((ga + 2) * BANK + b, b, sga)

            for b in range(BANK):
                process((ga + 1) * BANK + b, BANK + b,
                        (ga + 1) * BANK + b)

            @pl.when(ga + 3 <= GROUPS - 1)
            def _():
                for b in range(BANK):
                    fire((ga + 3) * BANK + b, BANK + b, sgb)

            return carry

        lax.fori_loop(0, (GROUPS - 1) // 2, body, 0)
        for b in range(BANK):
            process((GROUPS - 1) * BANK + b, b, (GROUPS - 1) * BANK + b)
        wait_store(PANELS - 2, 0)
        wait_store(PANELS - 1, 1)

    return gather_kernel


_RELAYOUT = _build_relayout()
_GATHER = _build_gather()


def kernel(item_id, table):
    idx_t = jnp.transpose(item_id).reshape(-1).astype(jnp.int32)
    table_t = jnp.transpose(table)
    tail = lax.slice(table, (TAIL0, 0), (V, D)).reshape(-1)
    table_lin = _RELAYOUT(table_t, tail).reshape(V, D)
    out1d = _GATHER(idx_t, table_lin)
    out5 = out1d.reshape(H, 4, B // 128, 8, 128)
    return out5.transpose(2, 4, 0, 1, 3).reshape(B, H, D)


# speedup vs baseline: 1.2342x; 1.0608x over previous
import functools

import jax
import jax.numpy as jnp
from jax import lax
from jax.experimental import pallas as pl
from jax.experimental.pallas import tpu as pltpu
from jax.experimental.pallas import tpu_sc as plsc

D = 32
B = 16384
H = 50
N = B * H
NC = 2
NS = 16
NW = NC * NS
TBW = B // (128 * NW)
PANELS = H * TBW
BANK = 8
GROUPS = PANELS // BANK

V = 1_000_000
FULL_TC = 7808
KPW = FULL_TC // NW
TAIL0 = FULL_TC * 128
TAILN = (V - TAIL0) * D

RP = D + 1
CP = 129


def _build_relayout():
    mesh = plsc.VectorSubcoreMesh(core_axis_name="c", subcore_axis_name="s")

    @functools.partial(
        pl.kernel,
        mesh=mesh,
        compiler_params=pltpu.CompilerParams(use_tc_tiling_on_sc=True,
                                             needs_layout_passes=False),
        out_type=jax.ShapeDtypeStruct((V * D,), jnp.float32),
        scratch_types=[
            pltpu.VMEM((D, 128), jnp.float32),
            pltpu.VMEM((D, 128), jnp.float32),
            pltpu.VMEM((128 * RP,), jnp.float32),
            pltpu.VMEM((128 * D,), jnp.float32),
            pltpu.VMEM((128 * D,), jnp.float32),
            pltpu.SemaphoreType.DMA,
            pltpu.SemaphoreType.DMA,
            pltpu.SemaphoreType.DMA,
            pltpu.SemaphoreType.DMA,
        ],
    )
    def relayout_kernel(table_t, tail, scratch, pan0, pan1, row_p, row0,
                        row1, si0, si1, so0, so1):
        wid = lax.axis_index("s") * NC + lax.axis_index("c")
        base = KPW * wid
        pans = (pan0, pan1)
        rows = (row0, row1)
        sins = (si0, si1)
        souts = (so0, so1)

        @pl.when(wid == 0)
        def _():
            def tbody(i, carry):
                pltpu.sync_copy(tail.at[pl.ds(i * 2048, 2048)],
                                row0.at[pl.ds(0, 2048)])
                pltpu.sync_copy(row0.at[pl.ds(0, 2048)],
                                scratch.at[pl.ds(TAIL0 * D + i * 2048,
                                                 2048)])
                return carry
            lax.fori_loop(0, TAILN // 2048, tbody, 0)

        ib = lax.iota(jnp.int32, 16)
        ibs = [(ib + 16 * jj) * RP for jj in range(8)]

        def fire_in(k, b):
            pltpu.async_copy(
                table_t.at[:, pl.ds((base + k) * 128, 128)], pans[b],
                sins[b])

        def wait_in(k, b):
            pltpu.make_async_copy(
                table_t.at[:, pl.ds((base + k) * 128, 128)], pans[b],
                sins[b]).wait()

        def fire_out(k, b):
            pltpu.async_copy(rows[b],
                             scratch.at[pl.ds((base + k) * 128 * D,
                                              128 * D)], souts[b])

        def wait_out(k, b):
            pltpu.make_async_copy(
                rows[b],
                scratch.at[pl.ds((base + k) * 128 * D, 128 * D)],
                souts[b]).wait()

        def transpose(pan, row):
            def dbody(dd, carry):
                for du in range(4):
                    d = dd * 4 + du
                    dvec = jnp.full((16,), 0, jnp.int32) + d
                    for jj in range(8):
                        v = pan[d, pl.ds(16 * jj, 16)]
                        plsc.store_scatter(row_p, [ibs[jj] + dvec], v)
                return carry
            lax.fori_loop(0, 8, dbody, 0)

            def cbody(cc, carry):
                for cl in range(8):
                    c = cc * 8 + cl
                    row[pl.ds(c * D, 16)] = row_p[pl.ds(c * RP, 16)]
                    row[pl.ds(c * D + 16, 16)] = (
                        row_p[pl.ds(c * RP + 16, 16)])
                return carry
            lax.fori_loop(0, 16, cbody, 0)

        fire_in(0, 0)
        fire_in(1, 1)

        def body(kk, carry):
            for par in range(2):
                k = 2 * kk + par
                wait_in(k, par)

                @pl.when(kk >= 1)
                def _():
                    wait_out(k - 2, par)

                transpose(pans[par], rows[par])
                fire_out(k, par)

                @pl.when(k + 2 <= KPW - 1)
                def _():
                    fire_in(k + 2, par)

            return carry

        lax.fori_loop(0, KPW // 2, body, 0)
        wait_out(KPW - 2, 0)
        wait_out(KPW - 1, 1)

    return relayout_kernel


def _build_gather():
    mesh = plsc.VectorSubcoreMesh(core_axis_name="c", subcore_axis_name="s")

    @functools.partial(
        pl.kernel,
        mesh=mesh,
        compiler_params=pltpu.CompilerParams(use_tc_tiling_on_sc=False,
                                             needs_layout_passes=False),
        out_type=jax.ShapeDtypeStruct((N, D), jnp.float32),
        scratch_types=[
            pltpu.VMEM((H, 128 * TBW), jnp.int32),
            pltpu.VMEM((2 * BANK, 128, D), jnp.float32),
            pltpu.SemaphoreType.DMA,
            pltpu.SemaphoreType.DMA,
            pltpu.SemaphoreType.DMA,
            pltpu.SemaphoreType.DMA,
            pltpu.SemaphoreType.DMA,
        ],
    )
    def gather_kernel(idx_hbm, table_hbm, out_hbm, idx_v, rows_v,
                      sga, sgb, sst0, sst1, sidx):
        wid = lax.axis_index("s") * NC + lax.axis_index("c")
        nbase = 128 * TBW * wid

        for h in range(H):
            pltpu.async_copy(
                idx_hbm.at[pl.ds(h * B + nbase, 128 * TBW)], idx_v.at[h],
                sidx)
        for h in range(H):
            pltpu.make_async_copy(
                idx_hbm.at[pl.ds(h * B + nbase, 128 * TBW)], idx_v.at[h],
                sidx).wait()

        ssts = (sst0, sst1)

        def fire(p, buf, sem):
            h = p // TBW
            t = p % TBW
            pltpu.async_copy(
                table_hbm.at[idx_v.at[h, pl.ds(t * 128, 128)]],
                rows_v.at[buf], sem)

        def wait_gather(p, buf, sem):
            h = p // TBW
            t = p % TBW
            pltpu.make_async_copy(
                table_hbm.at[idx_v.at[h, pl.ds(t * 128, 128)]],
                rows_v.at[buf], sem).wait()

        def store(p, buf):
            h = p // TBW
            t = p % TBW
            row0 = h * B + nbase + t * 128
            pltpu.async_copy(rows_v.at[buf],
                             out_hbm.at[pl.ds(row0, 128)],
                             ssts[buf % 2])

        def wait_store(p, buf):
            h = p // TBW
            t = p % TBW
            row0 = h * B + nbase + t * 128
            pltpu.make_async_copy(
                rows_v.at[buf],
                out_hbm.at[pl.ds(row0, 128)], ssts[buf % 2]).wait()

        def process(p, buf, pipe):
            wait_gather(p, buf, sga if buf < BANK else sgb)
            store(p, buf)

        for b in range(BANK):
            fire(b, b, sga)
        for b in range(BANK):
            fire(BANK + b, BANK + b, sgb)

        def body(gg, carry):
            ga = 2 * gg
            for b in range(BANK):
                process(ga * BANK + b, b, ga * BANK + b)

            @pl.when(ga + 2 <= GROUPS - 1)
            def _():
                for b in range(BANK):
                    wait_store(ga * BANK + b, b)
                    fire((ga + 2) * BANK + b, b, sga)

            for b in range(BANK):
                process((ga + 1) * BANK + b, BANK + b,
                        (ga + 1) * BANK + b)

            @pl.when(ga + 3 <= GROUPS - 1)
            def _():
                for b in range(BANK):
                    wait_store((ga + 1) * BANK + b, BANK + b)
                    fire((ga + 3) * BANK + b, BANK + b, sgb)

            return carry

        lax.fori_loop(0, (GROUPS - 1) // 2, body, 0)
        for b in range(BANK):
            process((GROUPS - 1) * BANK + b, b, (GROUPS - 1) * BANK + b)
        for b in range(BANK):
            wait_store((GROUPS - 2) * BANK + b, BANK + b)
        for b in range(BANK):
            wait_store((GROUPS - 1) * BANK + b, b)

    return gather_kernel


_RELAYOUT = _build_relayout()
_GATHER = _build_gather()


def kernel(item_id, table):
    idx_t = jnp.transpose(item_id).reshape(-1).astype(jnp.int32)
    table_t = jnp.transpose(table)
    tail = lax.slice(table, (TAIL0, 0), (V, D)).reshape(-1)
    table_lin = _RELAYOUT(table_t, tail).reshape(V, D)
    out2d = _GATHER(idx_t, table_lin)
    return out2d.reshape(H, B, D).transpose(1, 0, 2)


# speedup vs baseline: 1.3377x; 1.0839x over previous
import functools

import jax
import jax.numpy as jnp
from jax import lax
from jax.experimental import pallas as pl
from jax.experimental.pallas import tpu as pltpu
from jax.experimental.pallas import tpu_sc as plsc

D = 32
B = 16384
H = 50
N = B * H
NC = 2
NS = 16
NW = NC * NS
TBW = B // (128 * NW)
PANELS = H * TBW
BANK = 8
GROUPS = PANELS // BANK

V = 1_000_000
FULL_TC = 7808
KPW = FULL_TC // NW
TAIL0 = FULL_TC * 128
TAILN = (V - TAIL0) * D

RP = D + 1
CP = 129


def _build_relayout():
    mesh = plsc.VectorSubcoreMesh(core_axis_name="c", subcore_axis_name="s")

    @functools.partial(
        pl.kernel,
        mesh=mesh,
        compiler_params=pltpu.CompilerParams(use_tc_tiling_on_sc=True,
                                             needs_layout_passes=False),
        out_type=jax.ShapeDtypeStruct((V * D,), jnp.float32),
        scratch_types=[
            pltpu.VMEM((D, 128), jnp.float32),
            pltpu.VMEM((D, 128), jnp.float32),
            pltpu.VMEM((128 * RP,), jnp.float32),
            pltpu.VMEM((128 * D,), jnp.float32),
            pltpu.VMEM((128 * D,), jnp.float32),
            pltpu.SemaphoreType.DMA,
            pltpu.SemaphoreType.DMA,
            pltpu.SemaphoreType.DMA,
            pltpu.SemaphoreType.DMA,
        ],
    )
    def relayout_kernel(table_t, tail, scratch, pan0, pan1, row_p, row0,
                        row1, si0, si1, so0, so1):
        wid = lax.axis_index("s") * NC + lax.axis_index("c")
        base = KPW * wid
        pans = (pan0, pan1)
        rows = (row0, row1)
        sins = (si0, si1)
        souts = (so0, so1)

        @pl.when(wid == 0)
        def _():
            def tbody(i, carry):
                pltpu.sync_copy(tail.at[pl.ds(i * 2048, 2048)],
                                row0.at[pl.ds(0, 2048)])
                pltpu.sync_copy(row0.at[pl.ds(0, 2048)],
                                scratch.at[pl.ds(TAIL0 * D + i * 2048,
                                                 2048)])
                return carry
            lax.fori_loop(0, TAILN // 2048, tbody, 0)

        ib = lax.iota(jnp.int32, 16)
        ibs = [(ib + 16 * jj) * RP for jj in range(8)]

        def fire_in(k, b):
            pltpu.async_copy(
                table_t.at[:, pl.ds((base + k) * 128, 128)], pans[b],
                sins[b])

        def wait_in(k, b):
            pltpu.make_async_copy(
                table_t.at[:, pl.ds((base + k) * 128, 128)], pans[b],
                sins[b]).wait()

        def fire_out(k, b):
            pltpu.async_copy(rows[b],
                             scratch.at[pl.ds((base + k) * 128 * D,
                                              128 * D)], souts[b])

        def wait_out(k, b):
            pltpu.make_async_copy(
                rows[b],
                scratch.at[pl.ds((base + k) * 128 * D, 128 * D)],
                souts[b]).wait()

        def transpose(pan, row):
            for d in range(D):
                dvec = jnp.full((16,), d, jnp.int32)
                for jj in range(8):
                    v = pan[d, pl.ds(16 * jj, 16)]
                    plsc.store_scatter(row_p, [ibs[jj] + dvec], v)

            for c in range(128):
                row[pl.ds(c * D, 16)] = row_p[pl.ds(c * RP, 16)]
                row[pl.ds(c * D + 16, 16)] = row_p[pl.ds(c * RP + 16, 16)]

        fire_in(0, 0)
        fire_in(1, 1)

        def body(kk, carry):
            for par in range(2):
                k = 2 * kk + par
                wait_in(k, par)

                @pl.when(kk >= 1)
                def _():
                    wait_out(k - 2, par)

                transpose(pans[par], rows[par])
                fire_out(k, par)

                @pl.when(k + 2 <= KPW - 1)
                def _():
                    fire_in(k + 2, par)

            return carry

        lax.fori_loop(0, KPW // 2, body, 0)
        wait_out(KPW - 2, 0)
        wait_out(KPW - 1, 1)

    return relayout_kernel


def _build_gather():
    mesh = plsc.VectorSubcoreMesh(core_axis_name="c", subcore_axis_name="s")

    @functools.partial(
        pl.kernel,
        mesh=mesh,
        compiler_params=pltpu.CompilerParams(use_tc_tiling_on_sc=False,
                                             needs_layout_passes=False),
        out_type=jax.ShapeDtypeStruct((N, D), jnp.float32),
        scratch_types=[
            pltpu.VMEM((H, 128 * TBW), jnp.int32),
            pltpu.VMEM((2 * BANK, 128, D), jnp.float32),
            pltpu.SemaphoreType.DMA,
            pltpu.SemaphoreType.DMA,
            pltpu.SemaphoreType.DMA,
            pltpu.SemaphoreType.DMA,
            pltpu.SemaphoreType.DMA,
        ],
    )
    def gather_kernel(idx_hbm, table_hbm, out_hbm, idx_v, rows_v,
                      sga, sgb, sst0, sst1, sidx):
        wid = lax.axis_index("s") * NC + lax.axis_index("c")
        nbase = 128 * TBW * wid

        for h in range(H):
            pltpu.async_copy(
                idx_hbm.at[pl.ds(h * B + nbase, 128 * TBW)], idx_v.at[h],
                sidx)
        for h in range(H):
            pltpu.make_async_copy(
                idx_hbm.at[pl.ds(h * B + nbase, 128 * TBW)], idx_v.at[h],
                sidx).wait()

        ssts = (sst0, sst1)

        def fire(p, buf, sem):
            h = p // TBW
            t = p % TBW
            pltpu.async_copy(
                table_hbm.at[idx_v.at[h, pl.ds(t * 128, 128)]],
                rows_v.at[buf], sem)

        def wait_gather(p, buf, sem):
            h = p // TBW
            t = p % TBW
            pltpu.make_async_copy(
                table_hbm.at[idx_v.at[h, pl.ds(t * 128, 128)]],
                rows_v.at[buf], sem).wait()

        def store(p, buf):
            h = p // TBW
            t = p % TBW
            row0 = h * B + nbase + t * 128
            pltpu.async_copy(rows_v.at[buf],
                             out_hbm.at[pl.ds(row0, 128)],
                             ssts[buf % 2])

        def wait_store(p, buf):
            h = p // TBW
            t = p % TBW
            row0 = h * B + nbase + t * 128
            pltpu.make_async_copy(
                rows_v.at[buf],
                out_hbm.at[pl.ds(row0, 128)], ssts[buf % 2]).wait()

        def process(p, buf, pipe):
            wait_gather(p, buf, sga if buf < BANK else sgb)
            store(p, buf)

        for b in range(BANK):
            fire(b, b, sga)
        for b in range(BANK):
            fire(BANK + b, BANK + b, sgb)

        def body(gg, carry):
            ga = 2 * gg
            for b in range(BANK):
                process(ga * BANK + b, b, ga * BANK + b)

            @pl.when(ga + 2 <= GROUPS - 1)
            def _():
                for b in range(BANK):
                    wait_store(ga * BANK + b, b)
                    fire((ga + 2) * BANK + b, b, sga)

            for b in range(BANK):
                process((ga + 1) * BANK + b, BANK + b,
                        (ga + 1) * BANK + b)

            @pl.when(ga + 3 <= GROUPS - 1)
            def _():
                for b in range(BANK):
                    wait_store((ga + 1) * BANK + b, BANK + b)
                    fire((ga + 3) * BANK + b, BANK + b, sgb)

            return carry

        lax.fori_loop(0, (GROUPS - 1) // 2, body, 0)
        for b in range(BANK):
            process((GROUPS - 1) * BANK + b, b, (GROUPS - 1) * BANK + b)
        for b in range(BANK):
            wait_store((GROUPS - 2) * BANK + b, BANK + b)
        for b in range(BANK):
            wait_store((GROUPS - 1) * BANK + b, b)

    return gather_kernel


_RELAYOUT = _build_relayout()
_GATHER = _build_gather()


def kernel(item_id, table):
    idx_t = jnp.transpose(item_id).reshape(-1).astype(jnp.int32)
    table_t = jnp.transpose(table)
    tail = lax.slice(table, (TAIL0, 0), (V, D)).reshape(-1)
    table_lin = _RELAYOUT(table_t, tail).reshape(V, D)
    out2d = _GATHER(idx_t, table_lin)
    return out2d.reshape(H, B, D).transpose(1, 0, 2)


# speedup vs baseline: 1.4134x; 1.0566x over previous
import functools

import jax
import jax.numpy as jnp
from jax import lax
from jax.experimental import pallas as pl
from jax.experimental.pallas import tpu as pltpu
from jax.experimental.pallas import tpu_sc as plsc

D = 32
B = 16384
H = 50
N = B * H
NC = 2
NS = 16
NW = NC * NS
TBW = B // (128 * NW)
PANELS = H * TBW
BANK = 8
GROUPS = PANELS // BANK

V = 1_000_000
FULL_TC = 7808
KPW = FULL_TC // NW
TAIL0 = FULL_TC * 128
TAILN = (V - TAIL0) * D

RP = 40
CP = 129


def _build_relayout():
    mesh = plsc.VectorSubcoreMesh(core_axis_name="c", subcore_axis_name="s")

    @functools.partial(
        pl.kernel,
        mesh=mesh,
        compiler_params=pltpu.CompilerParams(use_tc_tiling_on_sc=True,
                                             needs_layout_passes=False),
        out_type=jax.ShapeDtypeStruct((V * RP,), jnp.float32),
        scratch_types=[
            pltpu.VMEM((D, 128), jnp.float32),
            pltpu.VMEM((D, 128), jnp.float32),
            pltpu.VMEM((128 * RP,), jnp.float32),
            pltpu.VMEM((128 * RP,), jnp.float32),
            pltpu.SemaphoreType.DMA,
            pltpu.SemaphoreType.DMA,
            pltpu.SemaphoreType.DMA,
            pltpu.SemaphoreType.DMA,
        ],
    )
    def relayout_kernel(table_t, tail, scratch, pan0, pan1, row0, row1,
                        si0, si1, so0, so1):
        wid = lax.axis_index("s") * NC + lax.axis_index("c")
        base = KPW * wid
        pans = (pan0, pan1)
        rows = (row0, row1)
        sins = (si0, si1)
        souts = (so0, so1)

        t0 = wid * 18
        pltpu.sync_copy(tail.at[pl.ds(t0 * D, 18 * D)],
                        row0.at[pl.ds(0, 18 * D)])
        for r in range(18):
            pltpu.async_copy(
                row0.at[pl.ds(r * D, D)],
                scratch.at[pl.ds((TAIL0 + t0 + r) * RP, D)], so0)
        for r in range(18):
            pltpu.make_async_copy(
                row0.at[pl.ds(r * D, D)],
                scratch.at[pl.ds((TAIL0 + t0 + r) * RP, D)], so0).wait()

        ib = lax.iota(jnp.int32, 16)
        ibs = [(ib + 16 * jj) * RP for jj in range(8)]

        def fire_in(k, b):
            pltpu.async_copy(
                table_t.at[:, pl.ds((base + k) * 128, 128)], pans[b],
                sins[b])

        def wait_in(k, b):
            pltpu.make_async_copy(
                table_t.at[:, pl.ds((base + k) * 128, 128)], pans[b],
                sins[b]).wait()

        def fire_out(k, b):
            pltpu.async_copy(rows[b],
                             scratch.at[pl.ds((base + k) * 128 * RP,
                                              128 * RP)], souts[b])

        def wait_out(k, b):
            pltpu.make_async_copy(
                rows[b],
                scratch.at[pl.ds((base + k) * 128 * RP, 128 * RP)],
                souts[b]).wait()

        def transpose(pan, row):
            for d in range(D):
                dvec = jnp.full((16,), d, jnp.int32)
                for jj in range(8):
                    v = pan[d, pl.ds(16 * jj, 16)]
                    plsc.store_scatter(row, [ibs[jj] + dvec], v)

        fire_in(0, 0)
        fire_in(1, 1)

        def body(kk, carry):
            for par in range(2):
                k = 2 * kk + par
                wait_in(k, par)

                @pl.when(kk >= 1)
                def _():
                    wait_out(k - 2, par)

                transpose(pans[par], rows[par])
                fire_out(k, par)

                @pl.when(k + 2 <= KPW - 1)
                def _():
                    fire_in(k + 2, par)

            return carry

        lax.fori_loop(0, KPW // 2, body, 0)
        wait_out(KPW - 2, 0)
        wait_out(KPW - 1, 1)

    return relayout_kernel


def _build_gather():
    mesh = plsc.VectorSubcoreMesh(core_axis_name="c", subcore_axis_name="s")

    @functools.partial(
        pl.kernel,
        mesh=mesh,
        compiler_params=pltpu.CompilerParams(use_tc_tiling_on_sc=False,
                                             needs_layout_passes=False),
        out_type=jax.ShapeDtypeStruct((N, D), jnp.float32),
        scratch_types=[
            pltpu.VMEM((H, 128 * TBW), jnp.int32),
            pltpu.VMEM((2 * BANK, 128, RP), jnp.float32),
            pltpu.SemaphoreType.DMA,
            pltpu.SemaphoreType.DMA,
            pltpu.SemaphoreType.DMA,
            pltpu.SemaphoreType.DMA,
            pltpu.SemaphoreType.DMA,
        ],
    )
    def gather_kernel(idx_hbm, table_hbm, out_hbm, idx_v, rows_v,
                      sga, sgb, sst0, sst1, sidx):
        wid = lax.axis_index("s") * NC + lax.axis_index("c")
        nbase = 128 * TBW * wid

        for h in range(H):
            pltpu.async_copy(
                idx_hbm.at[pl.ds(h * B + nbase, 128 * TBW)], idx_v.at[h],
                sidx)
        for h in range(H):
            pltpu.make_async_copy(
                idx_hbm.at[pl.ds(h * B + nbase, 128 * TBW)], idx_v.at[h],
                sidx).wait()

        ssts = (sst0, sst1)

        def fire(p, buf, sem):
            h = p // TBW
            t = p % TBW
            pltpu.async_copy(
                table_hbm.at[idx_v.at[h, pl.ds(t * 128, 128)]],
                rows_v.at[buf], sem)

        def wait_gather(p, buf, sem):
            h = p // TBW
            t = p % TBW
            pltpu.make_async_copy(
                table_hbm.at[idx_v.at[h, pl.ds(t * 128, 128)]],
                rows_v.at[buf], sem).wait()

        def store(p, buf):
            h = p // TBW
            t = p % TBW
            row0 = h * B + nbase + t * 128
            pltpu.async_copy(rows_v.at[buf, :, pl.ds(0, D)],
                             out_hbm.at[pl.ds(row0, 128)],
                             ssts[buf % 2])

        def wait_store(p, buf):
            h = p // TBW
            t = p % TBW
            row0 = h * B + nbase + t * 128
            pltpu.make_async_copy(
                rows_v.at[buf, :, pl.ds(0, D)],
                out_hbm.at[pl.ds(row0, 128)], ssts[buf % 2]).wait()

        def process(p, buf, pipe):
            wait_gather(p, buf, sga if buf < BANK else sgb)
            store(p, buf)

        for b in range(BANK):
            fire(b, b, sga)
        for b in range(BANK):
            fire(BANK + b, BANK + b, sgb)

        def body(gg, carry):
            ga = 2 * gg
            for b in range(BANK):
                process(ga * BANK + b, b, ga * BANK + b)

            @pl.when(ga + 2 <= GROUPS - 1)
            def _():
                for b in range(BANK):
                    wait_store(ga * BANK + b, b)
                    fire((ga + 2) * BANK + b, b, sga)

            for b in range(BANK):
                process((ga + 1) * BANK + b, BANK + b,
                        (ga + 1) * BANK + b)

            @pl.when(ga + 3 <= GROUPS - 1)
            def _():
                for b in range(BANK):
                    wait_store((ga + 1) * BANK + b, BANK + b)
                    fire((ga + 3) * BANK + b, BANK + b, sgb)

            return carry

        lax.fori_loop(0, (GROUPS - 1) // 2, body, 0)
        for b in range(BANK):
            process((GROUPS - 1) * BANK + b, b, (GROUPS - 1) * BANK + b)
        for b in range(BANK):
            wait_store((GROUPS - 2) * BANK + b, BANK + b)
        for b in range(BANK):
            wait_store((GROUPS - 1) * BANK + b, b)

    return gather_kernel


_RELAYOUT = _build_relayout()
_GATHER = _build_gather()


def kernel(item_id, table):
    idx_t = jnp.transpose(item_id).reshape(-1).astype(jnp.int32)
    table_t = jnp.transpose(table)
    tail = lax.slice(table, (TAIL0, 0), (V, D)).reshape(-1)
    table_lin = _RELAYOUT(table_t, tail).reshape(V, RP)
    out2d = _GATHER(idx_t, table_lin)
    return out2d.reshape(H, B, D).transpose(1, 0, 2)
